# baseline (speedup 1.0000x reference)
import functools

import jax
import jax.numpy as jnp
from jax import lax
from jax.experimental import pallas as pl
from jax.experimental.pallas import tpu as pltpu
from jax.experimental.pallas import tpu_sc as plsc

N = 10000
E = 320000
NODE_DIM = 128
EDGE_DIM = 16
H = 128

NC = 2
NS = 16
NW = NC * NS

SUB_S = 80
NCH_W = E // SUB_S // NW


def _gelu(x):
    return 0.5 * x * (1.0 + lax.erf(x * 0.7071067811865476))


def _bn(x, g, b, m, v, eps=1e-3):
    return (x - m) * (g * lax.rsqrt(v + eps)) + b


def _node_embed_body(x_ref, g1, b1, m1, v1, w1, c1, g2, b2, m2, v2, w2, c2,
                     o_ref):
    h = _bn(x_ref[...], g1[...], b1[...], m1[...], v1[...])
    h = _gelu(jnp.dot(h, w1[...], preferred_element_type=jnp.float32) + c1[...])
    h = _bn(h, g2[...], b2[...], m2[...], v2[...])
    h = _gelu(jnp.dot(h, w2[...], preferred_element_type=jnp.float32) + c2[...])
    o_ref[...] = h


def _node_embed_tc(x, p, block_rows):
    rows, d_in = x.shape
    grid = rows // block_rows
    vecs = [p[k].reshape(1, -1) for k in
            ("g1", "b1", "m1", "v1")] + [p["W1"], p["c1"].reshape(1, -1)] + \
           [p[k].reshape(1, -1) for k in ("g2", "b2", "m2", "v2")] + \
           [p["W2"], p["c2"].reshape(1, -1)]
    full = pl.BlockSpec(index_map=lambda i: (0, 0))
    in_specs = [pl.BlockSpec((block_rows, d_in), lambda i: (i, 0))] + \
               [full] * len(vecs)
    return pl.pallas_call(
        _node_embed_body,
        grid=(grid,),
        in_specs=in_specs,
        out_specs=pl.BlockSpec((block_rows, H), lambda i: (i, 0)),
        out_shape=jax.ShapeDtypeStruct((rows, H), jnp.float32),
    )(x, *vecs)


def _edge_mul_body(ea_ref, w1, b1, w2, b2, gath_ref, o_ref):
    h = _gelu(jnp.dot(ea_ref[...], w1[...], preferred_element_type=jnp.float32)
              + b1[...])
    h = _gelu(jnp.dot(h, w2[...], preferred_element_type=jnp.float32) + b2[...])
    o_ref[...] = h * gath_ref[...]


def _edge_embed_mul_tc(edge_attr, p, gathered, block_rows):
    grid = E // block_rows
    full = pl.BlockSpec(index_map=lambda i: (0, 0))
    return pl.pallas_call(
        _edge_mul_body,
        grid=(grid,),
        in_specs=[pl.BlockSpec((block_rows, EDGE_DIM), lambda i: (i, 0)),
                  full, full, full, full,
                  pl.BlockSpec((block_rows, H), lambda i: (i, 0))],
        out_specs=pl.BlockSpec((block_rows, H), lambda i: (i, 0)),
        out_shape=jax.ShapeDtypeStruct((E, H), jnp.float32),
    )(edge_attr, p["W1"], p["b1"].reshape(1, -1), p["W2"],
      p["b2"].reshape(1, -1), gathered)


def _final_body(x_ref, s_ref, c_ref, g1, b1, m1, v1, w1, c1, g2, b2, m2, v2,
                w2, c2, o_ref):
    sums = s_ref[0] + s_ref[1]
    cnt = (c_ref[0] + c_ref[1])[:, 0:1]
    agg = sums / jnp.maximum(cnt, 1.0)
    h = jnp.concatenate([x_ref[...], agg], axis=1)
    h = _bn(h, g1[...], b1[...], m1[...], v1[...])
    h = _gelu(jnp.dot(h, w1[...], preferred_element_type=jnp.float32) + c1[...])
    h = _bn(h, g2[...], b2[...], m2[...], v2[...])
    h = _gelu(jnp.dot(h, w2[...], preferred_element_type=jnp.float32) + c2[...])
    o_ref[...] = h


def _final_tc(x, part_sums, part_cnt, p, block_rows):
    grid = N // block_rows
    vecs = [p[k].reshape(1, -1) for k in
            ("g1", "b1", "m1", "v1")] + [p["W1"], p["c1"].reshape(1, -1)] + \
           [p[k].reshape(1, -1) for k in ("g2", "b2", "m2", "v2")] + \
           [p["W2"], p["c2"].reshape(1, -1)]
    full = pl.BlockSpec(index_map=lambda i: (0, 0))
    in_specs = [pl.BlockSpec((block_rows, NODE_DIM), lambda i: (i, 0)),
                pl.BlockSpec((NC, block_rows, H), lambda i: (0, i, 0)),
                pl.BlockSpec((NC, block_rows, H), lambda i: (0, i, 0))] + \
               [full] * len(vecs)
    return pl.pallas_call(
        _final_body,
        grid=(grid,),
        in_specs=in_specs,
        out_specs=pl.BlockSpec((block_rows, H), lambda i: (i, 0)),
        out_shape=jax.ShapeDtypeStruct((N, H), jnp.float32),
    )(x, part_sums, part_cnt, *vecs)


@functools.cache
def _sc_kernels():
    mesh = plsc.VectorSubcoreMesh(core_axis_name="c", subcore_axis_name="s",
                                  num_cores=NC, num_subcores=NS)

    NMB = N // SUB_S

    @functools.partial(
        pl.kernel,
        out_type=jax.ShapeDtypeStruct((E, H), jnp.float32),
        mesh=mesh,
        scratch_types=[
            pltpu.VMEM((NCH_W, SUB_S), jnp.int32),
            pltpu.VMEM((2, SUB_S, H), jnp.float32),
            pltpu.VMEM_SHARED((N, H), jnp.float32),
            pltpu.SemaphoreType.DMA,
        ],
    )
    def _sc_gather(nm_hbm, dstp_hbm, out_hbm, idx_v, rows_v, nm_sh, sem):
        cid = lax.axis_index("c")
        sid = lax.axis_index("s")
        wid = cid * NS + sid

        pltpu.sync_copy(dstp_hbm.at[wid], idx_v)

        def _stage(i, carry):
            k = i * NS + sid

            @pl.when(k < NMB)
            def _():
                pltpu.sync_copy(nm_hbm.at[pl.ds(k * SUB_S, SUB_S)],
                                nm_sh.at[pl.ds(k * SUB_S, SUB_S)])

            return carry

        lax.fori_loop(0, -(-NMB // NS), _stage, 0)
        plsc.subcore_barrier()

        pltpu.async_copy(nm_sh.at[idx_v.at[0]], rows_v.at[0], sem)

        def body(k, carry):
            b = k % 2

            @pl.when(k + 1 < NCH_W)
            def _():
                pltpu.async_copy(nm_sh.at[idx_v.at[k + 1]],
                                 rows_v.at[1 - b], sem)

            pltpu.make_async_copy(nm_hbm.at[pl.ds(0, SUB_S)],
                                  rows_v.at[b], sem).wait()
            e0 = (k * NW + wid) * SUB_S
            pltpu.sync_copy(rows_v.at[b], out_hbm.at[pl.ds(e0, SUB_S)])
            return carry

        lax.fori_loop(0, NCH_W, body, 0)

    @functools.partial(
        pl.kernel,
        out_type=jax.ShapeDtypeStruct((NC, N, H), jnp.float32),
        mesh=mesh,
        scratch_types=[
            pltpu.VMEM((NCH_W, SUB_S), jnp.int32),
            pltpu.VMEM((2, SUB_S, H), jnp.float32),
            pltpu.VMEM_SHARED((N, H), jnp.float32),
            pltpu.SemaphoreType.DMA,
        ],
    )
    def _sc_scatter(msgs_hbm, srcp_hbm, zrow_hbm,
                    out_s, idx_v, rows_v, acc_sh, sem):
        cid = lax.axis_index("c")
        sid = lax.axis_index("s")
        wid = cid * NS + sid

        pltpu.sync_copy(srcp_hbm.at[wid], idx_v)

        pltpu.sync_copy(zrow_hbm, rows_v.at[0])
        nzb = N // SUB_S

        def _zero(i, carry):
            k = i * NS + sid

            @pl.when(k < nzb)
            def _():
                pltpu.sync_copy(rows_v.at[0],
                                acc_sh.at[pl.ds(k * SUB_S, SUB_S)])

            return carry

        lax.fori_loop(0, -(-nzb // NS), _zero, 0)
        plsc.subcore_barrier()

        pltpu.async_copy(msgs_hbm.at[pl.ds(wid * SUB_S, SUB_S)],
                         rows_v.at[0], sem)

        def body(k, carry):
            b = k % 2

            @pl.when(k + 1 < NCH_W)
            def _():
                e1 = ((k + 1) * NW + wid) * SUB_S
                pltpu.async_copy(msgs_hbm.at[pl.ds(e1, SUB_S)],
                                 rows_v.at[1 - b], sem)

            pltpu.make_async_copy(msgs_hbm.at[pl.ds(0, SUB_S)],
                                  rows_v.at[b], sem).wait()
            pltpu.sync_copy(rows_v.at[b], acc_sh.at[idx_v.at[k]], add=True)
            return carry

        lax.fori_loop(0, NCH_W, body, 0)
        plsc.subcore_barrier()

        @pl.when(sid == 0)
        def _dump():
            pltpu.sync_copy(acc_sh, out_s.at[cid])

    NROWS_C = E // SUB_S
    ITERS_C = NROWS_C // NW
    NZB_C = N // SUB_S

    @functools.partial(
        pl.kernel,
        out_type=jax.ShapeDtypeStruct((NC, N, H), jnp.float32),
        mesh=mesh,
        scratch_types=[
            pltpu.VMEM((SUB_S,), jnp.int32),
            pltpu.VMEM((SUB_S, H), jnp.float32),
            pltpu.VMEM_SHARED((N, H), jnp.float32),
            pltpu.SemaphoreType.DMA,
        ],
    )
    def _sc_count(src_hbm, zrow_hbm, ones_hbm,
                  out_c, idx_v, ones_v, acc_sh, sem):
        cid = lax.axis_index("c")
        sid = lax.axis_index("s")
        wid = cid * NS + sid

        pltpu.sync_copy(zrow_hbm, ones_v)

        def _zero(i, carry):
            k = i * NS + sid

            @pl.when(k < NZB_C)
            def _():
                pltpu.sync_copy(ones_v, acc_sh.at[pl.ds(k * SUB_S, SUB_S)])

            return carry

        lax.fori_loop(0, -(-NZB_C // NS), _zero, 0)

        pltpu.sync_copy(ones_hbm, ones_v)
        plsc.subcore_barrier()

        def body(i, carry):
            r = i * NW + wid
            pltpu.sync_copy(src_hbm.at[pl.ds(r * SUB_S, SUB_S)], idx_v)
            pltpu.sync_copy(ones_v, acc_sh.at[idx_v], add=True)
            return carry

        lax.fori_loop(0, ITERS_C, body, 0)
        plsc.subcore_barrier()

        @pl.when(sid == 0)
        def _dump():
            pltpu.sync_copy(acc_sh, out_c.at[cid])

    return _sc_gather, _sc_scatter, _sc_count


def kernel(x, edge_index, edge_attr, params):
    src = edge_index[0]
    dst = edge_index[1]

    sc_gather, sc_scatter, sc_count = _sc_kernels()

    srcp = src.reshape(NCH_W, NW, SUB_S).transpose(1, 0, 2)
    dstp = dst.reshape(NCH_W, NW, SUB_S).transpose(1, 0, 2)

    nm = _node_embed_tc(x, params["bm"], block_rows=1000)

    gathered = sc_gather(nm, dstp)

    msgs = _edge_embed_mul_tc(edge_attr, params["et"], gathered,
                              block_rows=16000)

    zrow80 = jnp.zeros((SUB_S, H), jnp.float32)
    part_sums = sc_scatter(msgs, srcp, zrow80)

    ones80 = jnp.ones((SUB_S, H), jnp.float32)
    part_cnt = sc_count(src, zrow80, ones80)

    return _final_tc(x, part_sums, part_cnt, params["uf"], block_rows=1000)


# speedup vs baseline: 1.0134x; 1.0134x over previous
import functools

import jax
import jax.numpy as jnp
from jax import lax
from jax.experimental import pallas as pl
from jax.experimental.pallas import tpu as pltpu
from jax.experimental.pallas import tpu_sc as plsc

N = 10000
E = 320000
NODE_DIM = 128
EDGE_DIM = 16
H = 128

NC = 2
NS = 16
NW = NC * NS

SUB_S = 80
NCH_W = E // SUB_S // NW


def _gelu(x):
    return 0.5 * x * (1.0 + lax.erf(x * 0.7071067811865476))


def _bn(x, g, b, m, v, eps=1e-3):
    return (x - m) * (g * lax.rsqrt(v + eps)) + b


def _node_embed_body(x_ref, g1, b1, m1, v1, w1, c1, g2, b2, m2, v2, w2, c2,
                     o_ref):
    h = _bn(x_ref[...], g1[...], b1[...], m1[...], v1[...])
    h = _gelu(jnp.dot(h, w1[...], preferred_element_type=jnp.float32) + c1[...])
    h = _bn(h, g2[...], b2[...], m2[...], v2[...])
    h = _gelu(jnp.dot(h, w2[...], preferred_element_type=jnp.float32) + c2[...])
    o_ref[...] = h


def _node_embed_tc(x, p, block_rows):
    rows, d_in = x.shape
    grid = rows // block_rows
    vecs = [p[k].reshape(1, -1) for k in
            ("g1", "b1", "m1", "v1")] + [p["W1"], p["c1"].reshape(1, -1)] + \
           [p[k].reshape(1, -1) for k in ("g2", "b2", "m2", "v2")] + \
           [p["W2"], p["c2"].reshape(1, -1)]
    full = pl.BlockSpec(index_map=lambda i: (0, 0))
    in_specs = [pl.BlockSpec((block_rows, d_in), lambda i: (i, 0))] + \
               [full] * len(vecs)
    return pl.pallas_call(
        _node_embed_body,
        grid=(grid,),
        in_specs=in_specs,
        out_specs=pl.BlockSpec((block_rows, H), lambda i: (i, 0)),
        out_shape=jax.ShapeDtypeStruct((rows, H), jnp.float32),
    )(x, *vecs)


def _edge_mul_body(ea_ref, w1, b1, w2, b2, gath_ref, o_ref):
    h = _gelu(jnp.dot(ea_ref[...], w1[...], preferred_element_type=jnp.float32)
              + b1[...])
    h = _gelu(jnp.dot(h, w2[...], preferred_element_type=jnp.float32) + b2[...])
    o_ref[...] = h * gath_ref[...]


def _edge_embed_mul_tc(edge_attr, p, gathered, block_rows):
    grid = E // block_rows
    full = pl.BlockSpec(index_map=lambda i: (0, 0))
    return pl.pallas_call(
        _edge_mul_body,
        grid=(grid,),
        in_specs=[pl.BlockSpec((block_rows, EDGE_DIM), lambda i: (i, 0)),
                  full, full, full, full,
                  pl.BlockSpec((block_rows, H), lambda i: (i, 0))],
        out_specs=pl.BlockSpec((block_rows, H), lambda i: (i, 0)),
        out_shape=jax.ShapeDtypeStruct((E, H), jnp.float32),
    )(edge_attr, p["W1"], p["b1"].reshape(1, -1), p["W2"],
      p["b2"].reshape(1, -1), gathered)


def _final_body(x_ref, s_ref, c_ref, g1, b1, m1, v1, w1, c1, g2, b2, m2, v2,
                w2, c2, o_ref):
    sums = s_ref[0] + s_ref[1]
    cnt = (c_ref[0] + c_ref[1])[:, 0:1]
    agg = sums / jnp.maximum(cnt, 1.0)
    h = jnp.concatenate([x_ref[...], agg], axis=1)
    h = _bn(h, g1[...], b1[...], m1[...], v1[...])
    h = _gelu(jnp.dot(h, w1[...], preferred_element_type=jnp.float32) + c1[...])
    h = _bn(h, g2[...], b2[...], m2[...], v2[...])
    h = _gelu(jnp.dot(h, w2[...], preferred_element_type=jnp.float32) + c2[...])
    o_ref[...] = h


def _final_tc(x, part_sums, part_cnt, p, block_rows):
    grid = N // block_rows
    vecs = [p[k].reshape(1, -1) for k in
            ("g1", "b1", "m1", "v1")] + [p["W1"], p["c1"].reshape(1, -1)] + \
           [p[k].reshape(1, -1) for k in ("g2", "b2", "m2", "v2")] + \
           [p["W2"], p["c2"].reshape(1, -1)]
    full = pl.BlockSpec(index_map=lambda i: (0, 0))
    in_specs = [pl.BlockSpec((block_rows, NODE_DIM), lambda i: (i, 0)),
                pl.BlockSpec((NC, block_rows, H), lambda i: (0, i, 0)),
                pl.BlockSpec((NC, block_rows, H), lambda i: (0, i, 0))] + \
               [full] * len(vecs)
    return pl.pallas_call(
        _final_body,
        grid=(grid,),
        in_specs=in_specs,
        out_specs=pl.BlockSpec((block_rows, H), lambda i: (i, 0)),
        out_shape=jax.ShapeDtypeStruct((N, H), jnp.float32),
    )(x, part_sums, part_cnt, *vecs)


@functools.cache
def _sc_kernels():
    mesh = plsc.VectorSubcoreMesh(core_axis_name="c", subcore_axis_name="s",
                                  num_cores=NC, num_subcores=NS)

    NMB = N // SUB_S

    @functools.partial(
        pl.kernel,
        out_type=jax.ShapeDtypeStruct((E, H), jnp.float32),
        mesh=mesh,
        scratch_types=[
            pltpu.VMEM((NCH_W, SUB_S), jnp.int32),
            pltpu.VMEM((2, SUB_S, H), jnp.float32),
            pltpu.VMEM_SHARED((N, H), jnp.float32),
            pltpu.SemaphoreType.DMA,
        ],
    )
    def _sc_gather(nm_hbm, dstp_hbm, out_hbm, idx_v, rows_v, nm_sh, sem):
        cid = lax.axis_index("c")
        sid = lax.axis_index("s")
        wid = cid * NS + sid

        pltpu.sync_copy(dstp_hbm.at[wid], idx_v)

        def _stage(i, carry):
            k = i * NS + sid

            @pl.when(k < NMB)
            def _():
                pltpu.sync_copy(nm_hbm.at[pl.ds(k * SUB_S, SUB_S)],
                                nm_sh.at[pl.ds(k * SUB_S, SUB_S)])

            return carry

        lax.fori_loop(0, -(-NMB // NS), _stage, 0)
        plsc.subcore_barrier()

        pltpu.async_copy(nm_sh.at[idx_v.at[0]], rows_v.at[0], sem)

        def body(k, carry):
            b = k % 2

            @pl.when(k + 1 < NCH_W)
            def _():
                pltpu.async_copy(nm_sh.at[idx_v.at[k + 1]],
                                 rows_v.at[1 - b], sem)

            pltpu.make_async_copy(nm_hbm.at[pl.ds(0, SUB_S)],
                                  rows_v.at[b], sem).wait()
            e0 = (k * NW + wid) * SUB_S
            pltpu.sync_copy(rows_v.at[b], out_hbm.at[pl.ds(e0, SUB_S)])
            return carry

        lax.fori_loop(0, NCH_W, body, 0)

    @functools.partial(
        pl.kernel,
        out_type=jax.ShapeDtypeStruct((NC, N, H), jnp.float32),
        mesh=mesh,
        scratch_types=[
            pltpu.VMEM((NCH_W, SUB_S), jnp.int32),
            pltpu.VMEM((2, SUB_S, H), jnp.float32),
            pltpu.VMEM_SHARED((N, H), jnp.float32),
            pltpu.SemaphoreType.DMA,
        ],
    )
    def _sc_scatter(msgs_hbm, srcp_hbm, zrow_hbm,
                    out_s, idx_v, rows_v, acc_sh, sem):
        cid = lax.axis_index("c")
        sid = lax.axis_index("s")
        wid = cid * NS + sid

        pltpu.sync_copy(srcp_hbm.at[wid], idx_v)

        pltpu.sync_copy(zrow_hbm, rows_v.at[0])
        nzb = N // SUB_S

        def _zero(i, carry):
            k = i * NS + sid

            @pl.when(k < nzb)
            def _():
                pltpu.sync_copy(rows_v.at[0],
                                acc_sh.at[pl.ds(k * SUB_S, SUB_S)])

            return carry

        lax.fori_loop(0, -(-nzb // NS), _zero, 0)
        plsc.subcore_barrier()

        pltpu.async_copy(msgs_hbm.at[pl.ds(wid * SUB_S, SUB_S)],
                         rows_v.at[0], sem)

        def body(k, carry):
            b = k % 2

            @pl.when(k + 1 < NCH_W)
            def _():
                e1 = ((k + 1) * NW + wid) * SUB_S
                pltpu.async_copy(msgs_hbm.at[pl.ds(e1, SUB_S)],
                                 rows_v.at[1 - b], sem)

            pltpu.make_async_copy(msgs_hbm.at[pl.ds(0, SUB_S)],
                                  rows_v.at[b], sem).wait()
            pltpu.sync_copy(rows_v.at[b], acc_sh.at[idx_v.at[k]], add=True)
            return carry

        lax.fori_loop(0, NCH_W, body, 0)
        plsc.subcore_barrier()

        @pl.when(sid == 0)
        def _dump():
            pltpu.sync_copy(acc_sh, out_s.at[cid])

    NROWS_C = E // SUB_S
    ITERS_C = NROWS_C // NW
    NZB_C = N // SUB_S

    @functools.partial(
        pl.kernel,
        out_type=jax.ShapeDtypeStruct((NC, N, H), jnp.float32),
        mesh=mesh,
        scratch_types=[
            pltpu.VMEM((SUB_S,), jnp.int32),
            pltpu.VMEM((SUB_S, H), jnp.float32),
            pltpu.VMEM_SHARED((N, H), jnp.float32),
            pltpu.SemaphoreType.DMA,
        ],
    )
    def _sc_count(src_hbm, zrow_hbm, ones_hbm,
                  out_c, idx_v, ones_v, acc_sh, sem):
        cid = lax.axis_index("c")
        sid = lax.axis_index("s")
        wid = cid * NS + sid

        pltpu.sync_copy(zrow_hbm, ones_v)

        def _zero(i, carry):
            k = i * NS + sid

            @pl.when(k < NZB_C)
            def _():
                pltpu.sync_copy(ones_v, acc_sh.at[pl.ds(k * SUB_S, SUB_S)])

            return carry

        lax.fori_loop(0, -(-NZB_C // NS), _zero, 0)

        pltpu.sync_copy(ones_hbm, ones_v)
        plsc.subcore_barrier()

        def body(i, carry):
            r = i * NW + wid
            pltpu.sync_copy(src_hbm.at[pl.ds(r * SUB_S, SUB_S)], idx_v)
            pltpu.sync_copy(ones_v, acc_sh.at[idx_v], add=True)
            return carry

        lax.fori_loop(0, ITERS_C, body, 0)
        plsc.subcore_barrier()

        @pl.when(sid == 0)
        def _dump():
            pltpu.sync_copy(acc_sh, out_c.at[cid])

    return _sc_gather, _sc_scatter, _sc_count


def kernel(x, edge_index, edge_attr, params):
    src = edge_index[0]
    dst = edge_index[1]

    sc_gather, sc_scatter, sc_count = _sc_kernels()

    srcp = src.reshape(NCH_W, NW, SUB_S).transpose(1, 0, 2)
    dstp = dst.reshape(NCH_W, NW, SUB_S).transpose(1, 0, 2)

    nm = _node_embed_tc(x, params["bm"], block_rows=2000)

    gathered = sc_gather(nm, dstp)

    msgs = _edge_embed_mul_tc(edge_attr, params["et"], gathered,
                              block_rows=8000)

    zrow80 = jnp.zeros((SUB_S, H), jnp.float32)
    part_sums = sc_scatter(msgs, srcp, zrow80)

    ones80 = jnp.ones((SUB_S, H), jnp.float32)
    part_cnt = sc_count(src, zrow80, ones80)

    return _final_tc(x, part_sums, part_cnt, params["uf"], block_rows=1000)
